# Initial kernel scaffold; baseline (speedup 1.0000x reference)
import jax
import jax.numpy as jnp
from jax.experimental import pallas as pl


def kernel(x, edge_index, Wl1, Wr1, b1, Wl2, Wr2, b2, Wl3, Wr3, b3, Wl4, Wr4, b4):
    raise NotImplementedError("write your pallas kernel here")


# speedup vs baseline: 4.3814x; 4.3814x over previous
import functools

import jax
import jax.numpy as jnp
from jax import lax
from jax.experimental import pallas as pl
from jax.experimental.pallas import tpu as pltpu
from jax.experimental.pallas import tpu_sc as plsc

N = 10000
E = 320000
D = 128
NP = 10240
NC = 2
NS = 16
NW = NC * NS
CH = 128
NCHUNK = 79
EP = NW * NCHUNK * CH
RPT = NP // NS


def _mesh():
    return plsc.VectorSubcoreMesh(
        core_axis_name="c", subcore_axis_name="s", num_cores=NC, num_subcores=NS
    )


def _make_agg():
    def body(h_hbm, src_hbm, dst_hbm, zf_hbm, acc_out,
             src_v, dst_v, rows_v, acc_sh, sem):
        cid = lax.axis_index("c")
        sid = lax.axis_index("s")
        wid = cid * NS + sid

        pltpu.sync_copy(zf_hbm.at[pl.ds(sid * RPT, RPT)],
                        acc_sh.at[pl.ds(sid * RPT, RPT)])
        pltpu.sync_copy(src_hbm.at[wid], src_v)
        pltpu.sync_copy(dst_hbm.at[wid], dst_v)
        plsc.subcore_barrier()

        def chunk(c, carry):
            pltpu.async_copy(h_hbm.at[src_v.at[c]], rows_v, sem).wait()
            pltpu.sync_copy(rows_v, acc_sh.at[dst_v.at[c]], add=True)
            return carry

        lax.fori_loop(0, NCHUNK, chunk, 0)
        plsc.subcore_barrier()

        pltpu.sync_copy(acc_sh.at[pl.ds(sid * RPT, RPT)],
                        acc_out.at[cid, pl.ds(sid * RPT, RPT)])

    return pl.kernel(
        body,
        out_type=jax.ShapeDtypeStruct((NC, NP, D), jnp.float32),
        mesh=_mesh(),
        scratch_types=[
            pltpu.VMEM((NCHUNK, CH), jnp.int32),
            pltpu.VMEM((NCHUNK, CH), jnp.int32),
            pltpu.VMEM((CH, D), jnp.float32),
            pltpu.VMEM_SHARED((NP, D), jnp.float32),
            pltpu.SemaphoreType.DMA,
        ],
    )


def _make_cnt():
    def body(dst_hbm, zc_hbm, ones_hbm, cnt_out, dst_v, ones_v, cnt_sh):
        cid = lax.axis_index("c")
        sid = lax.axis_index("s")
        wid = cid * NS + sid

        pltpu.sync_copy(zc_hbm.at[pl.ds(sid * RPT, RPT)],
                        cnt_sh.at[pl.ds(sid * RPT, RPT)])
        pltpu.sync_copy(dst_hbm.at[wid], dst_v)
        pltpu.sync_copy(ones_hbm, ones_v)
        plsc.subcore_barrier()

        def chunk(c, carry):
            pltpu.sync_copy(ones_v, cnt_sh.at[dst_v.at[c]], add=True)
            return carry

        lax.fori_loop(0, NCHUNK, chunk, 0)
        plsc.subcore_barrier()

        pltpu.sync_copy(cnt_sh.at[pl.ds(sid * RPT, RPT)],
                        cnt_out.at[cid, pl.ds(sid * RPT, RPT)])

    return pl.kernel(
        body,
        out_type=jax.ShapeDtypeStruct((NC, NP, D), jnp.float32),
        mesh=_mesh(),
        scratch_types=[
            pltpu.VMEM((NCHUNK, CH), jnp.int32),
            pltpu.VMEM((CH, D), jnp.float32),
            pltpu.VMEM_SHARED((NP, D), jnp.float32),
        ],
    )


def _tc_layer(acc, cnt, h, wlT, wrT, b, act: str):
    BLK = 1024
    grid = (NP // BLK,)

    def body(acc_r, cnt_r, h_r, wl_r, wr_r, b_r, o_r):
        a = acc_r[0] + acc_r[1]
        c = cnt_r[0][:, :1] + cnt_r[1][:, :1]
        mean = a * (1.0 / jnp.maximum(c, 1.0))
        z = (jnp.dot(mean, wl_r[...], preferred_element_type=jnp.float32)
             + jnp.dot(h_r[...], wr_r[...], preferred_element_type=jnp.float32)
             + b_r[...])
        if act == "elu":
            o_r[...] = jnp.where(z > 0, z, jnp.exp(jnp.minimum(z, 0.0)) - 1.0)
        else:
            e = jnp.exp(-jnp.abs(z))
            o_r[...] = jnp.where(z >= 0, 1.0 / (1.0 + e), e / (1.0 + e))

    return pl.pallas_call(
        body,
        grid=grid,
        in_specs=[
            pl.BlockSpec((NC, BLK, D), lambda i: (0, i, 0)),
            pl.BlockSpec((NC, BLK, D), lambda i: (0, i, 0)),
            pl.BlockSpec((BLK, D), lambda i: (i, 0)),
            pl.BlockSpec((D, D), lambda i: (0, 0)),
            pl.BlockSpec((D, D), lambda i: (0, 0)),
            pl.BlockSpec((1, D), lambda i: (0, 0)),
        ],
        out_specs=pl.BlockSpec((BLK, D), lambda i: (i, 0)),
        out_shape=jax.ShapeDtypeStruct((NP, D), jnp.float32),
    )(acc, cnt, h, wlT, wrT, b)


def kernel(x, edge_index, Wl1, Wr1, b1, Wl2, Wr2, b2, Wl3, Wr3, b3, Wl4, Wr4, b4):
    f32 = jnp.float32
    pad_e = EP - E
    src = jnp.concatenate([edge_index[0], jnp.full((pad_e,), N, jnp.int32)])
    dst = jnp.concatenate([edge_index[1], jnp.full((pad_e,), N, jnp.int32)])
    src = src.reshape(NW, NCHUNK, CH)
    dst = dst.reshape(NW, NCHUNK, CH)

    h = jnp.zeros((NP, D), f32).at[:N].set(x.astype(f32))
    zf = jnp.zeros((NP, D), f32)
    ones = jnp.ones((CH, D), f32)

    wl4T = jnp.zeros((D, D), f32).at[:, :64].set(Wl4.T)
    wr4T = jnp.zeros((D, D), f32).at[:, :64].set(Wr4.T)
    b4p = jnp.zeros((1, D), f32).at[0, :64].set(b4)
    layers = [
        (Wl1.T, Wr1.T, b1.reshape(1, D), "elu"),
        (Wl2.T, Wr2.T, b2.reshape(1, D), "elu"),
        (Wl3.T, Wr3.T, b3.reshape(1, D), "elu"),
        (wl4T, wr4T, b4p, "sigmoid"),
    ]

    agg = _make_agg()
    cnt = _make_cnt()(dst, zf, ones)
    for wl, wr, b, act in layers:
        acc = agg(h, src, dst, zf)
        h = _tc_layer(acc, cnt, h, wl, wr, b, act)
    return h[:N, :64]
